# baseline (device time: 44248 ns/iter reference)
import jax
import jax.numpy as jnp
from jax import lax
from jax.experimental import pallas as pl
from jax.experimental.pallas import tpu as pltpu

N_DEV = 4


def kernel(x, Wg, Wu, Wd):
    m, d_in = x.shape
    d_out = Wd.shape[1]
    ch = m // N_DEV
    half = ch // 2

    def body(x_ref, wg_ref, wu_ref, wd_ref, out_ref,
             send_buf, rs_buf, own_buf,
             rs_send_sems, rs_recv_sems, ag_send_sems, ag_recv_sems):
        d = lax.axis_index("i")
        right = (d + 1) % N_DEV
        left = (d - 1) % N_DEV

        barrier_sem = pltpu.get_barrier_semaphore()
        for j in range(1, N_DEV):
            pl.semaphore_signal(
                barrier_sem, inc=1,
                device_id=((d + j) % N_DEV,),
                device_id_type=pl.DeviceIdType.MESH,
            )
        pl.semaphore_wait(barrier_sem, N_DEV - 1)

        wg = wg_ref[...]
        wu = wu_ref[...]
        wd = wd_ref[...]

        def partial_chunk(c):
            xc = x_ref[pl.ds(c * ch, ch), :]
            gate = jnp.dot(xc, wg, preferred_element_type=jnp.float32)
            up = jnp.dot(xc, wu, preferred_element_type=jnp.float32)
            h = gate * (up * jax.nn.sigmoid(up))
            return jnp.dot(h, wd, preferred_element_type=jnp.float32)

        rs = []
        for j in range(1, N_DEV):
            c = (d + j) % N_DEV
            send_buf[j - 1, :, :] = partial_chunk(c)
            rdma = pltpu.make_async_remote_copy(
                src_ref=send_buf.at[j - 1],
                dst_ref=rs_buf.at[N_DEV - 1 - j],
                send_sem=rs_send_sems.at[j - 1],
                recv_sem=rs_recv_sems.at[N_DEV - 1 - j],
                device_id=(c,),
                device_id_type=pl.DeviceIdType.MESH,
            )
            rdma.start()
            rs.append(rdma)

        acc = partial_chunk(d)
        for rdma in rs:
            rdma.wait_recv()
        acc = acc + rs_buf[0] + rs_buf[1] + rs_buf[2]
        own_buf[...] = acc
        out_ref[pl.ds(d * ch, ch), :] = acc

        def ag_copy(src, tgt, row_start, ssem, rsem):
            rdma = pltpu.make_async_remote_copy(
                src_ref=src,
                dst_ref=out_ref.at[pl.ds(row_start, half)],
                send_sem=ag_send_sems.at[ssem],
                recv_sem=ag_recv_sems.at[rsem],
                device_id=(tgt,),
                device_id_type=pl.DeviceIdType.MESH,
            )
            rdma.start()
            return rdma

        s1 = ag_copy(own_buf.at[pl.ds(0, half)], right, d * ch, 0, 0)
        s3 = ag_copy(own_buf.at[pl.ds(half, half)], left, d * ch + half, 2, 3)
        s2 = ag_copy(own_buf.at[pl.ds(half, half)], right, d * ch + half, 1, 1)
        s4 = ag_copy(own_buf.at[pl.ds(0, half)], left, d * ch, 3, 2)

        s1.wait_recv()
        r5 = ag_copy(out_ref.at[pl.ds(left * ch, half)], right, left * ch, 4, 4)
        s3.wait_recv()
        r6 = ag_copy(out_ref.at[pl.ds(right * ch + half, half)], left,
                     right * ch + half, 5, 5)

        for rdma in (s2, s4, r5, r6):
            rdma.wait_recv()
        for rdma in rs + [s1, s2, s3, s4, r5, r6]:
            rdma.wait_send()

    return pl.pallas_call(
        body,
        out_shape=jax.ShapeDtypeStruct((m, d_out), jnp.float32),
        in_specs=[pl.BlockSpec(memory_space=pltpu.VMEM)] * 4,
        out_specs=pl.BlockSpec(memory_space=pltpu.VMEM),
        scratch_shapes=[
            pltpu.VMEM((N_DEV - 1, ch, d_out), jnp.float32),
            pltpu.VMEM((N_DEV - 1, ch, d_out), jnp.float32),
            pltpu.VMEM((ch, d_out), jnp.float32),
            pltpu.SemaphoreType.DMA((N_DEV - 1,)),
            pltpu.SemaphoreType.DMA((N_DEV - 1,)),
            pltpu.SemaphoreType.DMA((6,)),
            pltpu.SemaphoreType.DMA((6,)),
        ],
        compiler_params=pltpu.CompilerParams(collective_id=0),
    )(x, Wg, Wu, Wd)


# device time: 33433 ns/iter; 1.3235x vs baseline; 1.3235x over previous
import jax
import jax.numpy as jnp
from jax import lax
from jax.experimental import pallas as pl
from jax.experimental.pallas import tpu as pltpu

N_DEV = 4


def kernel(x, Wg, Wu, Wd):
    m, d_in = x.shape
    d_out = Wd.shape[1]
    ch = m // N_DEV
    half = ch // 2

    def body(x_ref, wg_ref, wu_ref, wd_ref, out_ref,
             send_buf, rs_buf, own_buf, ag_buf,
             rs_send_sems, rs_recv_sems, ag_send_sems, ag_recv_sems):
        d = lax.axis_index("i")
        right = (d + 1) % N_DEV
        left = (d - 1) % N_DEV
        pf = 3 - d
        pn = d ^ 1
        z = (d + 2) % N_DEV

        barrier_sem = pltpu.get_barrier_semaphore()
        for nbr in (left, right):
            pl.semaphore_signal(
                barrier_sem, inc=1,
                device_id=(nbr,), device_id_type=pl.DeviceIdType.MESH,
            )
        pl.semaphore_wait(barrier_sem, 2)

        wg = wg_ref[...]
        wu = wu_ref[...]
        wd = wd_ref[...]

        def partial_chunk(c):
            xc = x_ref[pl.ds(c * ch, ch), :]
            gate = jnp.dot(xc, wg, preferred_element_type=jnp.float32)
            up = jnp.dot(xc, wu, preferred_element_type=jnp.float32)
            h = gate * (up * jax.nn.sigmoid(up))
            return jnp.dot(h, wd, preferred_element_type=jnp.float32)

        def rs_copy(src_slot, dst_slot, tgt):
            rdma = pltpu.make_async_remote_copy(
                src_ref=send_buf.at[src_slot],
                dst_ref=rs_buf.at[dst_slot],
                send_sem=rs_send_sems.at[src_slot],
                recv_sem=rs_recv_sems.at[dst_slot],
                device_id=(tgt,),
                device_id_type=pl.DeviceIdType.MESH,
            )
            rdma.start()
            return rdma

        send_buf[0, :, :] = partial_chunk(z).astype(jnp.bfloat16)
        r1a = rs_copy(0, 0, pf)
        send_buf[1, :, :] = partial_chunk(pf).astype(jnp.bfloat16)
        r1b = rs_copy(1, 1, pf)
        c3 = partial_chunk(pn)
        r1a.wait_recv()
        send_buf[2, :, :] = (
            c3 + rs_buf[0].astype(jnp.float32)
        ).astype(jnp.bfloat16)
        r2 = rs_copy(2, 2, pn)
        acc = partial_chunk(d)
        r1b.wait_recv()
        r2.wait_recv()
        acc = (acc + rs_buf[1].astype(jnp.float32)
               + rs_buf[2].astype(jnp.float32))
        own_buf[...] = acc.astype(jnp.bfloat16)
        out_ref[pl.ds(d * ch, ch), :] = acc

        def ag_copy(src, tgt, slot, h0, ssem, rsem):
            rdma = pltpu.make_async_remote_copy(
                src_ref=src,
                dst_ref=ag_buf.at[slot, pl.ds(h0, half)],
                send_sem=ag_send_sems.at[ssem],
                recv_sem=ag_recv_sems.at[rsem],
                device_id=(tgt,),
                device_id_type=pl.DeviceIdType.MESH,
            )
            rdma.start()
            return rdma

        s1 = ag_copy(own_buf.at[pl.ds(0, half)], right, 0, 0, 0, 0)
        s3 = ag_copy(own_buf.at[pl.ds(half, half)], left, 1, half, 2, 3)
        s2 = ag_copy(own_buf.at[pl.ds(half, half)], right, 0, half, 1, 1)
        s4 = ag_copy(own_buf.at[pl.ds(0, half)], left, 1, 0, 3, 2)

        s1.wait_recv()
        r5 = ag_copy(ag_buf.at[0, pl.ds(0, half)], right, 2, 0, 4, 4)
        s3.wait_recv()
        r6 = ag_copy(ag_buf.at[1, pl.ds(half, half)], left, 2, half, 5, 5)

        s2.wait_recv()
        out_ref[pl.ds(left * ch, ch), :] = ag_buf[0].astype(jnp.float32)
        s4.wait_recv()
        out_ref[pl.ds(right * ch, ch), :] = ag_buf[1].astype(jnp.float32)
        r5.wait_recv()
        r6.wait_recv()
        out_ref[pl.ds(z * ch, ch), :] = ag_buf[2].astype(jnp.float32)

        for rdma in (r1a, r1b, r2, s1, s2, s3, s4, r5, r6):
            rdma.wait_send()

    return pl.pallas_call(
        body,
        out_shape=jax.ShapeDtypeStruct((m, d_out), jnp.float32),
        in_specs=[pl.BlockSpec(memory_space=pltpu.VMEM)] * 4,
        out_specs=pl.BlockSpec(memory_space=pltpu.VMEM),
        scratch_shapes=[
            pltpu.VMEM((N_DEV - 1, ch, d_out), jnp.bfloat16),
            pltpu.VMEM((N_DEV - 1, ch, d_out), jnp.bfloat16),
            pltpu.VMEM((ch, d_out), jnp.bfloat16),
            pltpu.VMEM((N_DEV - 1, ch, d_out), jnp.bfloat16),
            pltpu.SemaphoreType.DMA((N_DEV - 1,)),
            pltpu.SemaphoreType.DMA((N_DEV - 1,)),
            pltpu.SemaphoreType.DMA((6,)),
            pltpu.SemaphoreType.DMA((6,)),
        ],
        compiler_params=pltpu.CompilerParams(collective_id=0),
    )(x, Wg, Wu, Wd)
